# baseline (device time: 108657 ns/iter reference)
import jax
import jax.numpy as jnp
from jax import lax
from jax.experimental import pallas as pl
from jax.experimental.pallas import tpu as pltpu

N_DEV = 8
B = 2
SQ_SH = 128
SKV = 1024
HQ = 4
DH = 64
D_MODEL = 512
BLK = 64


def kernel(x, Wq, K_ext, V_ext, Wo):
    def body(x_ref, wq_ref, k_ref, v_ref, wo_ref, out_ref,
             kfull_ref, vfull_ref, comm_ref, send_sems, recv_sems):
        my = lax.axis_index("i")
        left = lax.rem(my + N_DEV - 1, N_DEV)
        right = lax.rem(my + 1, N_DEV)

        barrier_sem = pltpu.get_barrier_semaphore()
        for nbr in (left, right):
            pl.semaphore_signal(
                barrier_sem, inc=1,
                device_id=(nbr,), device_id_type=pl.DeviceIdType.MESH,
            )
        pl.semaphore_wait(barrier_sem, 2)

        kfull_ref[:, pl.ds(my * SQ_SH, SQ_SH), :, :] = k_ref[...]
        vfull_ref[:, pl.ds(my * SQ_SH, SQ_SH), :, :] = v_ref[...]
        comm_ref[0, 0] = k_ref[...]
        comm_ref[0, 1] = v_ref[...]

        for h in range(N_DEV - 1):
            send_slot = h % 2
            recv_slot = (h + 1) % 2
            rdma = pltpu.make_async_remote_copy(
                src_ref=comm_ref.at[send_slot],
                dst_ref=comm_ref.at[recv_slot],
                send_sem=send_sems.at[send_slot],
                recv_sem=recv_sems.at[recv_slot],
                device_id=(right,),
                device_id_type=pl.DeviceIdType.MESH,
            )
            rdma.start()
            rdma.wait()

            origin = lax.rem(my + 2 * N_DEV - h - 1, N_DEV)
            kfull_ref[:, pl.ds(origin * SQ_SH, SQ_SH), :, :] = comm_ref[recv_slot, 0]
            vfull_ref[:, pl.ds(origin * SQ_SH, SQ_SH), :, :] = comm_ref[recv_slot, 1]

        qb = (my * SQ_SH + lax.broadcasted_iota(jnp.int32, (SQ_SH, SKV), 0)) // BLK
        kb = lax.broadcasted_iota(jnp.int32, (SQ_SH, SKV), 1) // BLK
        mask = (qb == kb) | (kb == 0) | (lax.rem(qb + kb, 3) == 0)

        wq = wq_ref[...]
        wo = wo_ref[...]
        for b in range(B):
            q_b = jnp.dot(x_ref[b], wq, preferred_element_type=jnp.float32)
            ctx_heads = []
            for hd in range(HQ):
                q_h = q_b[:, hd * DH:(hd + 1) * DH]
                k_h = kfull_ref[b, :, hd, :]
                v_h = vfull_ref[b, :, hd, :]
                s = lax.dot_general(
                    q_h, k_h, (((1,), (1,)), ((), ())),
                    preferred_element_type=jnp.float32,
                ) * 0.125
                s = jnp.where(mask, s, -1e9)
                m = jnp.max(s, axis=-1, keepdims=True)
                w = jnp.exp(s - m)
                w = w / jnp.sum(w, axis=-1, keepdims=True)
                ctx_heads.append(
                    jnp.dot(w, v_h, preferred_element_type=jnp.float32)
                )
            ctx = jnp.concatenate(ctx_heads, axis=-1)
            out_ref[b] = jnp.dot(ctx, wo, preferred_element_type=jnp.float32)

    return pl.pallas_call(
        body,
        out_shape=jax.ShapeDtypeStruct((B, SQ_SH, D_MODEL), jnp.float32),
        in_specs=[pl.BlockSpec(memory_space=pltpu.VMEM)] * 5,
        out_specs=pl.BlockSpec(memory_space=pltpu.VMEM),
        scratch_shapes=[
            pltpu.VMEM((B, SKV, HQ, DH), jnp.float32),
            pltpu.VMEM((B, SKV, HQ, DH), jnp.float32),
            pltpu.VMEM((2, 2, B, SQ_SH, HQ, DH), jnp.float32),
            pltpu.SemaphoreType.DMA((2,)),
            pltpu.SemaphoreType.DMA((2,)),
        ],
        compiler_params=pltpu.CompilerParams(collective_id=0),
    )(x, Wq, K_ext, V_ext, Wo)


# device time: 61791 ns/iter; 1.7585x vs baseline; 1.7585x over previous
import jax
import jax.numpy as jnp
from jax import lax
from jax.experimental import pallas as pl
from jax.experimental.pallas import tpu as pltpu

N_DEV = 8
B = 2
SQ = 128
HQ = 4
DH = 64
D_MODEL = 512
BLK = 64
N_RG = N_DEV // 2
N_LG = N_DEV - 1 - N_RG


def kernel(x, Wq, K_ext, V_ext, Wo):
    def body(x_ref, wq_ref, k_ref, v_ref, wo_ref, out_ref,
             mine_ref, rg_ref, lg_ref, sr_sems, sl_sems, rr_sems, rl_sems):
        my = lax.axis_index("i")
        left = lax.rem(my + N_DEV - 1, N_DEV)
        right = lax.rem(my + 1, N_DEV)

        barrier_sem = pltpu.get_barrier_semaphore()
        for nbr in (left, right):
            pl.semaphore_signal(
                barrier_sem, inc=1,
                device_id=(nbr,), device_id_type=pl.DeviceIdType.MESH,
            )
        pl.semaphore_wait(barrier_sem, 2)

        def copy(src, dst, send_sem, recv_sem, dev):
            return pltpu.make_async_remote_copy(
                src_ref=src, dst_ref=dst, send_sem=send_sem,
                recv_sem=recv_sem, device_id=(dev,),
                device_id_type=pl.DeviceIdType.MESH,
            )

        mine_ref[0] = k_ref[...]
        mine_ref[1] = v_ref[...]
        send_r0 = copy(mine_ref, rg_ref.at[0], sr_sems.at[0], rr_sems.at[0], right)
        send_l0 = copy(mine_ref, lg_ref.at[0], sl_sems.at[0], rl_sems.at[0], left)
        send_r0.start()
        send_l0.start()

        q = [jnp.dot(x_ref[b], wq_ref[...], preferred_element_type=jnp.float32)
             for b in range(B)]

        def chunk_mask(origin):
            qb = my * 2 + lax.broadcasted_iota(jnp.int32, (SQ, SQ), 0) // BLK
            kb = origin * 2 + lax.broadcasted_iota(jnp.int32, (SQ, SQ), 1) // BLK
            return (qb == kb) | (kb == 0) | (lax.rem(qb + kb, 3) == 0)

        def masked_scores(qh, kh, msk):
            s = lax.dot_general(
                qh, kh, (((1,), (1,)), ((), ())),
                preferred_element_type=jnp.float32,
            ) * 0.125
            return jnp.where(msk, s, -1e9)

        msk0 = chunk_mask(my)
        m = [[None] * HQ for _ in range(B)]
        l = [[None] * HQ for _ in range(B)]
        acc = [[None] * HQ for _ in range(B)]
        for b in range(B):
            for h in range(HQ):
                qh = q[b][:, h * DH:(h + 1) * DH]
                s = masked_scores(qh, k_ref[b, :, h, :], msk0)
                mi = jnp.max(s, axis=-1, keepdims=True)
                p = jnp.exp(s - mi)
                m[b][h] = mi
                l[b][h] = jnp.sum(p, axis=-1, keepdims=True)
                acc[b][h] = jnp.dot(
                    p, v_ref[b, :, h, :], preferred_element_type=jnp.float32)

        def update(kv_ref, slot, origin):
            msk = chunk_mask(origin)
            for b in range(B):
                for h in range(HQ):
                    qh = q[b][:, h * DH:(h + 1) * DH]
                    s = masked_scores(qh, kv_ref[slot, 0, b, :, h, :], msk)
                    m_new = jnp.maximum(
                        m[b][h], jnp.max(s, axis=-1, keepdims=True))
                    p = jnp.exp(s - m_new)
                    sc = jnp.exp(m[b][h] - m_new)
                    m[b][h] = m_new
                    l[b][h] = l[b][h] * sc + jnp.sum(p, axis=-1, keepdims=True)
                    acc[b][h] = acc[b][h] * sc + jnp.dot(
                        p, kv_ref[slot, 1, b, :, h, :],
                        preferred_element_type=jnp.float32)

        for s_i in range(N_RG):
            copy(rg_ref.at[s_i], rg_ref.at[s_i],
                 rr_sems.at[s_i], rr_sems.at[s_i], left).wait_recv()
            if s_i < N_RG - 1:
                copy(rg_ref.at[s_i], rg_ref.at[s_i + 1],
                     sr_sems.at[s_i + 1], rr_sems.at[s_i + 1], right).start()
            if s_i < N_LG:
                copy(lg_ref.at[s_i], lg_ref.at[s_i],
                     rl_sems.at[s_i], rl_sems.at[s_i], right).wait_recv()
                if s_i < N_LG - 1:
                    copy(lg_ref.at[s_i], lg_ref.at[s_i + 1],
                         sl_sems.at[s_i + 1], rl_sems.at[s_i + 1], left).start()
            update(rg_ref, s_i, lax.rem(my + 2 * N_DEV - 1 - s_i, N_DEV))
            if s_i < N_LG:
                update(lg_ref, s_i, lax.rem(my + 1 + s_i, N_DEV))

        wo = wo_ref[...]
        for b in range(B):
            ctx = jnp.concatenate(
                [acc[b][h] / l[b][h] for h in range(HQ)], axis=-1)
            out_ref[b] = jnp.dot(ctx, wo, preferred_element_type=jnp.float32)

        send_r0.wait_send()
        send_l0.wait_send()
        for s_i in range(1, N_RG):
            copy(rg_ref.at[s_i - 1], rg_ref.at[s_i],
                 sr_sems.at[s_i], rr_sems.at[s_i], right).wait_send()
        for s_i in range(1, N_LG):
            copy(lg_ref.at[s_i - 1], lg_ref.at[s_i],
                 sl_sems.at[s_i], rl_sems.at[s_i], left).wait_send()

    chunk = (2, B, SQ, HQ, DH)
    return pl.pallas_call(
        body,
        out_shape=jax.ShapeDtypeStruct((B, SQ, D_MODEL), jnp.float32),
        in_specs=[pl.BlockSpec(memory_space=pltpu.VMEM)] * 5,
        out_specs=pl.BlockSpec(memory_space=pltpu.VMEM),
        scratch_shapes=[
            pltpu.VMEM(chunk, jnp.float32),
            pltpu.VMEM((N_RG,) + chunk, jnp.float32),
            pltpu.VMEM((N_LG,) + chunk, jnp.float32),
            pltpu.SemaphoreType.DMA((N_RG,)),
            pltpu.SemaphoreType.DMA((N_LG,)),
            pltpu.SemaphoreType.DMA((N_RG,)),
            pltpu.SemaphoreType.DMA((N_LG,)),
        ],
        compiler_params=pltpu.CompilerParams(collective_id=0),
    )(x, Wq, K_ext, V_ext, Wo)


# device time: 60797 ns/iter; 1.7872x vs baseline; 1.0163x over previous
import jax
import jax.numpy as jnp
from jax import lax
from jax.experimental import pallas as pl
from jax.experimental.pallas import tpu as pltpu

N_DEV = 8
B = 2
SQ = 128
HQ = 4
DH = 64
D_MODEL = 512
BLK = 64
BH = B * HQ
N_RG = N_DEV // 2
N_LG = N_DEV - 1 - N_RG


def kernel(x, Wq, K_ext, V_ext, Wo):
    def body(x_ref, wq_ref, k_ref, v_ref, wo_ref, out_ref,
             mine_ref, rg_ref, lg_ref, sr_sems, sl_sems, rr_sems, rl_sems):
        my = lax.axis_index("i")
        left = lax.rem(my + N_DEV - 1, N_DEV)
        right = lax.rem(my + 1, N_DEV)

        barrier_sem = pltpu.get_barrier_semaphore()
        for nbr in (left, right):
            pl.semaphore_signal(
                barrier_sem, inc=1,
                device_id=(nbr,), device_id_type=pl.DeviceIdType.MESH,
            )
        pl.semaphore_wait(barrier_sem, 2)

        def copy(src, dst, send_sem, recv_sem, dev):
            return pltpu.make_async_remote_copy(
                src_ref=src, dst_ref=dst, send_sem=send_sem,
                recv_sem=recv_sem, device_id=(dev,),
                device_id_type=pl.DeviceIdType.MESH,
            )

        for b in range(B):
            for h in range(HQ):
                mine_ref[0, b, h] = k_ref[b, :, h, :]
                mine_ref[1, b, h] = v_ref[b, :, h, :]
        send_r0 = copy(mine_ref, rg_ref.at[0], sr_sems.at[0], rr_sems.at[0], right)
        send_l0 = copy(mine_ref, lg_ref.at[0], sl_sems.at[0], rl_sems.at[0], left)
        send_r0.start()
        send_l0.start()

        q_all = jnp.stack([
            jnp.dot(x_ref[b], wq_ref[...], preferred_element_type=jnp.float32)
            .reshape(SQ, HQ, DH).transpose(1, 0, 2)
            for b in range(B)
        ]).reshape(BH, SQ, DH)

        def chunk_mask(origin):
            qb = my * 2 + lax.broadcasted_iota(jnp.int32, (SQ, SQ), 0) // BLK
            kb = origin * 2 + lax.broadcasted_iota(jnp.int32, (SQ, SQ), 1) // BLK
            return (qb == kb) | (kb == 0) | (lax.rem(qb + kb, 3) == 0)

        def chunk_scores(k_all, origin):
            s = lax.dot_general(
                q_all, k_all, (((2,), (2,)), ((0,), (0,))),
                preferred_element_type=jnp.float32,
            ) * 0.125
            return jnp.where(chunk_mask(origin)[None], s, -1e9)

        def pv(p, v_all):
            return lax.dot_general(
                p, v_all, (((2,), (1,)), ((0,), (0,))),
                preferred_element_type=jnp.float32,
            )

        s = chunk_scores(mine_ref[0].reshape(BH, SQ, DH), my)
        m = jnp.max(s, axis=-1, keepdims=True)
        p = jnp.exp(s - m)
        l = jnp.sum(p, axis=-1, keepdims=True)
        acc = pv(p, mine_ref[1].reshape(BH, SQ, DH))

        def update(kv_ref, slot, origin):
            nonlocal m, l, acc
            s = chunk_scores(kv_ref[slot, 0].reshape(BH, SQ, DH), origin)
            m_new = jnp.maximum(m, jnp.max(s, axis=-1, keepdims=True))
            p = jnp.exp(s - m_new)
            sc = jnp.exp(m - m_new)
            m = m_new
            l = l * sc + jnp.sum(p, axis=-1, keepdims=True)
            acc = acc * sc + pv(p, kv_ref[slot, 1].reshape(BH, SQ, DH))

        for s_i in range(N_RG):
            copy(rg_ref.at[s_i], rg_ref.at[s_i],
                 rr_sems.at[s_i], rr_sems.at[s_i], left).wait_recv()
            if s_i < N_RG - 1:
                copy(rg_ref.at[s_i], rg_ref.at[s_i + 1],
                     sr_sems.at[s_i + 1], rr_sems.at[s_i + 1], right).start()
            if s_i < N_LG:
                copy(lg_ref.at[s_i], lg_ref.at[s_i],
                     rl_sems.at[s_i], rl_sems.at[s_i], right).wait_recv()
                if s_i < N_LG - 1:
                    copy(lg_ref.at[s_i], lg_ref.at[s_i + 1],
                         sl_sems.at[s_i + 1], rl_sems.at[s_i + 1], left).start()
            update(rg_ref, s_i, lax.rem(my + 2 * N_DEV - 1 - s_i, N_DEV))
            if s_i < N_LG:
                update(lg_ref, s_i, lax.rem(my + 1 + s_i, N_DEV))

        o = acc / l
        wo = wo_ref[...]
        for b in range(B):
            ctx = jnp.concatenate(
                [o[b * HQ + h] for h in range(HQ)], axis=-1)
            out_ref[b] = jnp.dot(ctx, wo, preferred_element_type=jnp.float32)

        send_r0.wait_send()
        send_l0.wait_send()
        for s_i in range(1, N_RG):
            copy(rg_ref.at[s_i - 1], rg_ref.at[s_i],
                 sr_sems.at[s_i], rr_sems.at[s_i], right).wait_send()
        for s_i in range(1, N_LG):
            copy(lg_ref.at[s_i - 1], lg_ref.at[s_i],
                 sl_sems.at[s_i], rl_sems.at[s_i], left).wait_send()

    chunk = (2, B, HQ, SQ, DH)
    return pl.pallas_call(
        body,
        out_shape=jax.ShapeDtypeStruct((B, SQ, D_MODEL), jnp.float32),
        in_specs=[pl.BlockSpec(memory_space=pltpu.VMEM)] * 5,
        out_specs=pl.BlockSpec(memory_space=pltpu.VMEM),
        scratch_shapes=[
            pltpu.VMEM(chunk, jnp.float32),
            pltpu.VMEM((N_RG,) + chunk, jnp.float32),
            pltpu.VMEM((N_LG,) + chunk, jnp.float32),
            pltpu.SemaphoreType.DMA((N_RG,)),
            pltpu.SemaphoreType.DMA((N_LG,)),
            pltpu.SemaphoreType.DMA((N_RG,)),
            pltpu.SemaphoreType.DMA((N_LG,)),
        ],
        compiler_params=pltpu.CompilerParams(collective_id=0),
    )(x, Wq, K_ext, V_ext, Wo)


# device time: 40527 ns/iter; 2.6811x vs baseline; 1.5002x over previous
import jax
import jax.numpy as jnp
from jax import lax
from jax.experimental import pallas as pl
from jax.experimental.pallas import tpu as pltpu

N_DEV = 8
B = 2
SQ = 128
HQ = 4
DH = 64
D_MODEL = 512
BLK = 64


def kernel(x, Wq, K_ext, V_ext, Wo):
    def body(x_ref, wq_ref, k_ref, v_ref, wo_ref, out_ref,
             mine_ref, r1_ref, r2_ref, s1_sems, s2_sems, r1_sems, r2_sems):
        my = lax.axis_index("i")
        pp = lax.rem(my, 4)
        base = my - pp
        obase = 4 - base
        x1 = pp + 1 - 2 * lax.rem(pp, 2)
        x3 = 3 - pp
        x2 = lax.rem(pp + 2, 4)
        dx = base + x1
        dy = base + x3
        dz = lax.rem(my + 4, N_DEV)
        ox, oy, oxy = base + x1, base + x3, base + x2
        oz, oxz, oyz, oxyz = obase + pp, obase + x1, obase + x3, obase + x2

        barrier_sem = pltpu.get_barrier_semaphore()
        for nbr in (dx, dy, dz):
            pl.semaphore_signal(
                barrier_sem, inc=1,
                device_id=(nbr,), device_id_type=pl.DeviceIdType.MESH,
            )
        pl.semaphore_wait(barrier_sem, 3)

        def copy(src, dst, send_sem, recv_sem, dev):
            return pltpu.make_async_remote_copy(
                src_ref=src, dst_ref=dst, send_sem=send_sem,
                recv_sem=recv_sem, device_id=(dev,),
                device_id_type=pl.DeviceIdType.MESH,
            )

        for b in range(B):
            for h in range(HQ):
                mine_ref[b, 0, h] = k_ref[b, :, h, :]
                mine_ref[b, 1, h] = v_ref[b, :, h, :]

        copy(mine_ref.at[0], r1_ref.at[0], s1_sems.at[0], r1_sems.at[0], dx).start()
        copy(mine_ref.at[1], r2_ref.at[0], s2_sems.at[0], r2_sems.at[0], dz).start()
        copy(mine_ref.at[0], r1_ref.at[1], s1_sems.at[1], r1_sems.at[1], dy).start()
        copy(mine_ref.at[1], r2_ref.at[1], s2_sems.at[1], r2_sems.at[1], dy).start()
        copy(mine_ref.at[1], r2_ref.at[3], s2_sems.at[2], r2_sems.at[3], dx).start()
        copy(mine_ref.at[0], r1_ref.at[3], s1_sems.at[2], r1_sems.at[3], dz).start()

        q_all = jnp.stack([
            jnp.dot(x_ref[b], wq_ref[...], preferred_element_type=jnp.float32)
            .reshape(SQ, HQ, DH).transpose(1, 0, 2)
            for b in range(B)
        ])

        def chunk_mask(origin):
            qb = my * 2 + lax.broadcasted_iota(jnp.int32, (SQ, SQ), 0) // BLK
            kb = origin * 2 + lax.broadcasted_iota(jnp.int32, (SQ, SQ), 1) // BLK
            return (qb == kb) | (kb == 0) | (lax.rem(qb + kb, 3) == 0)

        def chunk_scores(qh, k_all, origin):
            s = lax.dot_general(
                qh, k_all, (((2,), (2,)), ((0,), (0,))),
                preferred_element_type=jnp.float32,
            ) * 0.125
            return jnp.where(chunk_mask(origin)[None], s, -1e9)

        def pv(p, v_all):
            return lax.dot_general(
                p, v_all, (((2,), (1,)), ((0,), (0,))),
                preferred_element_type=jnp.float32,
            )

        def recv_wait(ref, sems, slot):
            copy(ref.at[slot], ref.at[slot], sems.at[slot], sems.at[slot],
                 dx).wait_recv()

        recv_wait(r1_ref, r1_sems, 0)
        copy(r1_ref.at[0], r1_ref.at[2], s1_sems.at[3], r1_sems.at[2], dy).start()
        copy(r1_ref.at[0], r1_ref.at[4], s1_sems.at[4], r1_sems.at[4], dz).start()
        recv_wait(r2_ref, r2_sems, 0)
        copy(r2_ref.at[0], r2_ref.at[2], s2_sems.at[3], r2_sems.at[2], dy).start()
        copy(r2_ref.at[0], r2_ref.at[4], s2_sems.at[4], r2_sems.at[4], dx).start()
        recv_wait(r1_ref, r1_sems, 1)
        copy(r1_ref.at[1], r1_ref.at[5], s1_sems.at[5], r1_sems.at[5], dz).start()
        recv_wait(r2_ref, r2_sems, 1)
        copy(r2_ref.at[1], r2_ref.at[5], s2_sems.at[5], r2_sems.at[5], dx).start()

        m, l, acc = [None, None], [None, None], [None, None]
        for b in range(B):
            s = chunk_scores(q_all[b], mine_ref[b, 0], my)
            m[b] = jnp.max(s, axis=-1, keepdims=True)
            p = jnp.exp(s - m[b])
            l[b] = jnp.sum(p, axis=-1, keepdims=True)
            acc[b] = pv(p, mine_ref[b, 1])

        def update(b, kv_ref, slot, origin):
            s = chunk_scores(q_all[b], kv_ref[slot, 0], origin)
            m_new = jnp.maximum(m[b], jnp.max(s, axis=-1, keepdims=True))
            p = jnp.exp(s - m_new)
            sc = jnp.exp(m[b] - m_new)
            m[b] = m_new
            l[b] = l[b] * sc + jnp.sum(p, axis=-1, keepdims=True)
            acc[b] = acc[b] * sc + pv(p, kv_ref[slot, 1])

        update(0, r1_ref, 0, ox)
        update(1, r2_ref, 0, oz)
        update(0, r1_ref, 1, oy)
        update(1, r2_ref, 1, oy)

        recv_wait(r1_ref, r1_sems, 3)
        update(0, r1_ref, 3, oz)
        recv_wait(r2_ref, r2_sems, 3)
        update(1, r2_ref, 3, ox)
        recv_wait(r1_ref, r1_sems, 2)
        copy(r1_ref.at[2], r1_ref.at[6], s1_sems.at[6], r1_sems.at[6], dz).start()
        update(0, r1_ref, 2, oxy)
        recv_wait(r2_ref, r2_sems, 2)
        copy(r2_ref.at[2], r2_ref.at[6], s2_sems.at[6], r2_sems.at[6], dx).start()
        update(1, r2_ref, 2, oyz)
        recv_wait(r1_ref, r1_sems, 4)
        update(0, r1_ref, 4, oxz)
        recv_wait(r2_ref, r2_sems, 4)
        update(1, r2_ref, 4, oxz)
        recv_wait(r1_ref, r1_sems, 5)
        update(0, r1_ref, 5, oyz)
        recv_wait(r2_ref, r2_sems, 5)
        update(1, r2_ref, 5, oxy)
        recv_wait(r1_ref, r1_sems, 6)
        update(0, r1_ref, 6, oxyz)
        recv_wait(r2_ref, r2_sems, 6)
        update(1, r2_ref, 6, oxyz)

        wo = wo_ref[...]
        for b in range(B):
            o = acc[b] / l[b]
            ctx = jnp.concatenate([o[h] for h in range(HQ)], axis=-1)
            out_ref[b] = jnp.dot(ctx, wo, preferred_element_type=jnp.float32)

        for i in range(7):
            copy(r1_ref.at[0], r1_ref.at[0], s1_sems.at[i], r1_sems.at[0],
                 dx).wait_send()
            copy(r2_ref.at[0], r2_ref.at[0], s2_sems.at[i], r2_sems.at[0],
                 dx).wait_send()

    half = (2, HQ, SQ, DH)
    return pl.pallas_call(
        body,
        out_shape=jax.ShapeDtypeStruct((B, SQ, D_MODEL), jnp.float32),
        in_specs=[pl.BlockSpec(memory_space=pltpu.VMEM)] * 5,
        out_specs=pl.BlockSpec(memory_space=pltpu.VMEM),
        scratch_shapes=[
            pltpu.VMEM((B,) + half, jnp.float32),
            pltpu.VMEM((7,) + half, jnp.float32),
            pltpu.VMEM((7,) + half, jnp.float32),
            pltpu.SemaphoreType.DMA((7,)),
            pltpu.SemaphoreType.DMA((7,)),
            pltpu.SemaphoreType.DMA((7,)),
            pltpu.SemaphoreType.DMA((7,)),
        ],
        compiler_params=pltpu.CompilerParams(collective_id=0),
    )(x, Wq, K_ext, V_ext, Wo)
